# Initial kernel scaffold; baseline (speedup 1.0000x reference)
import jax
import jax.numpy as jnp
from jax.experimental import pallas as pl


def kernel(skill_ids, stats, skill_emb, proj_W, proj_b):
    raise NotImplementedError("write your pallas kernel here")


# speedup vs baseline: 18.1032x; 18.1032x over previous
import functools

import jax
import jax.numpy as jnp
from jax import lax
from jax.experimental import pallas as pl
from jax.experimental.pallas import tpu as pltpu
from jax.experimental.pallas import tpu_sc as plsc

_B = 16384
_L = 50
_D = 16
_ST = 5
_OUT = 64

_NC, _NS = 2, 16
_NW = _NC * _NS
_RPW = _B // _NW
_CH = 64
_NCHUNK = _RPW // _CH
_GI = _CH * _L


def _sc_pool(ids_flat, table, zrs, seg):
    mesh = plsc.VectorSubcoreMesh(core_axis_name="c", subcore_axis_name="s")

    @functools.partial(
        pl.kernel,
        mesh=mesh,
        compiler_params=pltpu.CompilerParams(use_tc_tiling_on_sc=False),
        out_type=jax.ShapeDtypeStruct((_B, _D), jnp.float32),
        scratch_types=[
            pltpu.VMEM((_GI,), jnp.int32),
            pltpu.VMEM((_GI,), jnp.int32),
            pltpu.VMEM((_GI, _D), jnp.float32),
            pltpu.VMEM_SHARED((_NS * _CH, _D), jnp.float32),
            pltpu.SemaphoreType.DMA,
        ],
    )
    def body(ids_hbm, table_hbm, z_hbm, seg_hbm, out_hbm,
             idx_v, seg_v, rows_v, pool_s, sem):
        sid = lax.axis_index("s")
        wid = sid * _NC + lax.axis_index("c")
        pltpu.sync_copy(seg_hbm.at[pl.ds(sid * _GI, _GI)], seg_v)

        def chunk(c, carry):
            base = wid * _RPW + c * _CH
            pltpu.sync_copy(ids_hbm.at[pl.ds(base * _L, _GI)], idx_v)
            pltpu.sync_copy(z_hbm, pool_s.at[pl.ds(sid * _CH, _CH)])
            pltpu.async_copy(table_hbm.at[idx_v], rows_v, sem).wait()
            pltpu.sync_copy(rows_v, pool_s.at[seg_v], add=True)
            pltpu.sync_copy(pool_s.at[pl.ds(sid * _CH, _CH)],
                            out_hbm.at[pl.ds(base, _CH)])
            return carry

        lax.fori_loop(0, _NCHUNK, chunk, 0)

    return body(ids_flat, table, zrs, seg)


def _tc_head(ids, sums, stats, w, b):
    rows = 2048
    grid = (_B // rows,)

    def body(ids_ref, sums_ref, stats_ref, w_ref, b_ref, out_ref):
        idv = ids_ref[...]
        cnt = jnp.sum((idv != 0).astype(jnp.float32), axis=1, keepdims=True)
        denom = jnp.maximum(cnt, 1.0)
        pooled = sums_ref[...] / denom
        wm = w_ref[...]
        acc = lax.dot_general(pooled, wm[:, :_D], (((1,), (1,)), ((), ())),
                              preferred_element_type=jnp.float32)
        acc = acc + lax.dot_general(stats_ref[...], wm[:, _D:],
                                    (((1,), (1,)), ((), ())),
                                    preferred_element_type=jnp.float32)
        out_ref[...] = jnp.maximum(acc + b_ref[...], 0.0)

    return pl.pallas_call(
        body,
        grid=grid,
        in_specs=[
            pl.BlockSpec((rows, _L), lambda i: (i, 0)),
            pl.BlockSpec((rows, _D), lambda i: (i, 0)),
            pl.BlockSpec((rows, _ST), lambda i: (i, 0)),
            pl.BlockSpec((_OUT, _D + _ST), lambda i: (0, 0)),
            pl.BlockSpec((1, _OUT), lambda i: (0, 0)),
        ],
        out_specs=pl.BlockSpec((rows, _OUT), lambda i: (i, 0)),
        out_shape=jax.ShapeDtypeStruct((_B, _OUT), jnp.float32),
    )(ids, sums, stats, w, b.reshape(1, _OUT))


def kernel(skill_ids, stats, skill_emb, proj_W, proj_b):
    ids_flat = skill_ids.reshape(_B * _L)
    seg = (jnp.arange(_GI, dtype=jnp.int32) // _L)[None, :] + (
        jnp.arange(_NS, dtype=jnp.int32) * _CH)[:, None]
    seg = seg.reshape(_NS * _GI)
    zrs = jnp.zeros((_CH, _D), jnp.float32)
    sums = _sc_pool(ids_flat, skill_emb, zrs, seg)
    return _tc_head(skill_ids, sums, stats, proj_W, proj_b)
